# baseline (speedup 1.0000x reference)
import functools

import jax
import jax.numpy as jnp
from jax import lax
from jax.experimental import pallas as pl
from jax.experimental.pallas import tpu as pltpu
from jax.experimental.pallas import tpu_sc as plsc


def _make_sc_gather(d: int, n_rows: int):
    info = plsc.get_sparse_core_info()
    nc, ns = info.num_cores, info.num_subcores
    nw = nc * ns
    b_per_w = n_rows // nw
    mesh = plsc.VectorSubcoreMesh(core_axis_name="c", subcore_axis_name="s")

    @functools.partial(
        pl.kernel,
        mesh=mesh,
        out_type=jax.ShapeDtypeStruct((n_rows, d), jnp.float32),
        scratch_types=[
            pltpu.VMEM((b_per_w,), jnp.int32),
            pltpu.VMEM((b_per_w, d), jnp.float32),
            pltpu.SemaphoreType.DMA,
        ],
    )
    def gather_kernel(table_hbm, idx_hbm, out_hbm, idx_v, rows_v, sem):
        wid = lax.axis_index("s") * nc + lax.axis_index("c")
        base = wid * b_per_w
        pltpu.sync_copy(idx_hbm.at[pl.ds(base, b_per_w)], idx_v)
        pltpu.async_copy(table_hbm.at[idx_v], rows_v, sem).wait()
        pltpu.sync_copy(rows_v, out_hbm.at[pl.ds(base, b_per_w)])

    return gather_kernel


def _lane_shuffle(x, idx16):
    dnums = lax.GatherDimensionNumbers(
        offset_dims=(), collapsed_slice_dims=(0,), start_index_map=(0,))
    return lax.gather(x, idx16[:, None], dnums, (1,),
                      mode=lax.GatherScatterMode.PROMISE_IN_BOUNDS)


def _make_sc_score(b_start: int, b_sc: int, n_neg: int, d: int):
    info = plsc.get_sparse_core_info()
    nc, ns = info.num_cores, info.num_subcores
    nw = nc * ns
    b_per_w = b_sc // nw
    chunk = 8
    n_chunks = b_per_w // chunk
    mesh = plsc.VectorSubcoreMesh(core_axis_name="c", subcore_axis_name="s")

    @functools.partial(
        pl.kernel,
        mesh=mesh,
        out_type=jax.ShapeDtypeStruct((b_sc, n_neg), jnp.float32),
        scratch_types=[
            pltpu.VMEM((b_per_w,), jnp.int32),
            pltpu.VMEM((b_per_w, d), jnp.float32),
            pltpu.VMEM((b_per_w, d), jnp.float32),
            pltpu.VMEM((chunk, n_neg, d), jnp.float32),
            pltpu.VMEM((b_per_w, n_neg), jnp.float32),
            pltpu.SemaphoreType.DMA,
        ],
    )
    def score_kernel(head_hbm, tail_hbm, idx_hbm, table_hbm, out_hbm,
                     idx_v, rel_v, hr_v, tail_v, score_v, sem):
        wid = lax.axis_index("s") * nc + lax.axis_index("c")
        wbase = b_start + wid * b_per_w
        obase = wid * b_per_w
        pltpu.sync_copy(idx_hbm.at[pl.ds(wbase, b_per_w)], idx_v)
        pltpu.async_copy(table_hbm.at[idx_v], rel_v, sem).wait()
        pltpu.sync_copy(head_hbm.at[pl.ds(wbase, b_per_w)], hr_v)

        def hr_body(r, carry):
            for j in range(d // 16):
                sl = pl.ds(j * 16, 16)
                hr_v[r, sl] = hr_v[r, sl] * rel_v[r, sl]
            return carry

        lax.fori_loop(0, b_per_w, hr_body, 0)

        lanes = lax.broadcasted_iota(jnp.int32, (16,), 0)
        perms = [jnp.bitwise_and(lanes + sh, 15) for sh in (8, 4, 2, 1)]

        def chunk_body(c, carry):
            pltpu.sync_copy(tail_hbm.at[pl.ds(wbase + c * chunk, chunk)], tail_v)

            def b_body(bb, carry2):
                brow = c * chunk + bb

                def ng_body(ng, carry3):
                    row16 = jnp.zeros((16,), jnp.float32)
                    for r in range(16):
                        n = ng * 16 + r
                        acc = tail_v[bb, n, pl.ds(0, 16)] * hr_v[brow, pl.ds(0, 16)]
                        for j in range(1, d // 16):
                            sl = pl.ds(j * 16, 16)
                            acc = acc + tail_v[bb, n, sl] * hr_v[brow, sl]
                        for p in perms:
                            acc = acc + _lane_shuffle(acc, p)
                        row16 = jnp.where(lanes == r, acc, row16)
                    score_v[brow, pl.ds(ng * 16, 16)] = row16
                    return carry3

                lax.fori_loop(0, n_neg // 16, ng_body, 0)
                return carry2

            lax.fori_loop(0, chunk, b_body, 0)
            return carry

        lax.fori_loop(0, n_chunks, chunk_body, 0)
        pltpu.sync_copy(score_v, out_hbm.at[pl.ds(obase, b_per_w)])

    return score_kernel


def _score_body(head_ref, rel_ref, tail_ref, out_ref):
    hr = head_ref[...] * rel_ref[...]
    out_ref[...] = jnp.sum(tail_ref[...] * hr[:, None, :], axis=2)


def kernel(head_embs, tail_embs, rel_idx, w_relation):
    b, n_neg, d = tail_embs.shape
    b_sc = 1024
    b_tc = b - b_sc
    idx = rel_idx.astype(jnp.int32)

    score_sc = _make_sc_score(b_tc, b_sc, n_neg, d)(
        head_embs, tail_embs, idx, w_relation)

    rel_tc = _make_sc_gather(d, b_tc)(w_relation, idx)

    blk = 512
    score_tc = pl.pallas_call(
        _score_body,
        grid=(b_tc // blk,),
        in_specs=[
            pl.BlockSpec((blk, d), lambda i: (i, 0)),
            pl.BlockSpec((blk, d), lambda i: (i, 0)),
            pl.BlockSpec((blk, n_neg, d), lambda i: (i, 0, 0)),
        ],
        out_specs=pl.BlockSpec((blk, n_neg), lambda i: (i, 0)),
        out_shape=jax.ShapeDtypeStruct((b_tc, n_neg), jnp.float32),
    )(head_embs, rel_tc, tail_embs)

    return jnp.concatenate([score_tc, score_sc], axis=0)
